# baseline (speedup 1.0000x reference)
import jax
import jax.numpy as jnp
import numpy as np
from jax import lax
from jax.experimental import pallas as pl
from jax.experimental.pallas import tpu as pltpu
from jax.experimental.pallas import tpu_sc as plsc

_NC = 2
_NS = 16
_NW = _NC * _NS
_L = 16

_B, _C, _H, _W = 8, 96, 512, 128
_P = _B * _C
_PS = 256
_PPW = _PS // _NW
_HALF = 32768
_OHALF = 8192
_NH = 2 * _PPW
_GROUPS = _HALF // 512


def _half_stencil(in_v, out_v):
    iota = lax.iota(jnp.int32, _L)
    ev = iota * 2

    consts = []
    for jb in range(4):
        ce = ev + (32 * jb)
        co = ce + 1
        consts.append((ce, co, co + 128, co + 256,
                       ce + 128, ce + 256, ce + 384, co + 384))

    @plsc.parallel_loop(0, _GROUPS, unroll=2)
    def row_body(i):
        grp = in_v.at[pl.ds(i * 512, 512)]
        for jb in range(4):
            ce, co, co1, co2, ce1, ce2, ce3, co3 = consts[jb]
            a = plsc.load_gather(grp, [ce])
            b = plsc.load_gather(grp, [co])
            c = plsc.load_gather(grp, [co1])
            d = plsc.load_gather(grp, [co2])
            out_v[pl.ds(i * 128 + 16 * jb, _L)] = jnp.maximum(
                jnp.maximum(a, b), jnp.maximum(c, d))
            e = plsc.load_gather(grp, [ce1])
            f = plsc.load_gather(grp, [ce2])
            g = plsc.load_gather(grp, [ce3])
            h = plsc.load_gather(grp, [co3])
            out_v[pl.ds(i * 128 + 64 + 16 * jb, _L)] = jnp.maximum(
                jnp.maximum(e, f), jnp.maximum(g, h))


def _sc_body(in_hbm, out_hbm, in_bufs, out_bufs, in_sems, out_sems):
    wid = lax.axis_index("s") * _NC + lax.axis_index("c")
    ibase = wid * (_NH * _HALF)
    obase = wid * (_NH * _OHALF)

    def in_slice(g):
        return in_hbm.at[pl.ds(ibase + g * _HALF, _HALF)]

    def out_slice(g):
        return out_hbm.at[pl.ds(obase + g * _OHALF, _OHALF)]

    for b in range(2):
        pltpu.async_copy(in_slice(b), in_bufs[b], in_sems[b])

    def pair_body(g2, carry):
        for b in range(2):
            g = g2 * 2 + b
            in_v, out_v = in_bufs[b], out_bufs[b]
            pltpu.make_async_copy(in_slice(g), in_v, in_sems[b]).wait()

            @pl.when(g2 > 0)
            def _wait_out():
                pltpu.make_async_copy(out_v, out_slice(g - 2),
                                      out_sems[b]).wait()

            _half_stencil(in_v, out_v)
            pltpu.async_copy(out_v, out_slice(g), out_sems[b])

            @pl.when(g2 < _NH // 2 - 1)
            def _prefetch():
                pltpu.async_copy(in_slice(g + 2), in_v, in_sems[b])
        return carry

    lax.fori_loop(0, _NH // 2, pair_body, 0)

    for b in range(2):
        pltpu.make_async_copy(out_bufs[b], out_slice(_NH - 2 + b),
                              out_sems[b]).wait()


_sc_pool = pl.kernel(
    _sc_body,
    out_type=jax.ShapeDtypeStruct((_PS * _H * _W // 4,), jnp.float32),
    mesh=plsc.VectorSubcoreMesh(core_axis_name="c", subcore_axis_name="s"),
    scratch_types=[
        [pltpu.VMEM((_HALF,), jnp.float32) for _ in range(2)],
        [pltpu.VMEM((_OHALF,), jnp.float32) for _ in range(2)],
        [pltpu.SemaphoreType.DMA for _ in range(2)],
        [pltpu.SemaphoreType.DMA for _ in range(2)],
    ],
    compiler_params=pltpu.CompilerParams(needs_layout_passes=False),
)


def _tc_body(x_ref, sel_ref, o_ref):
    x = x_ref[0]
    s = pltpu.roll(x, _W - 1, 1)
    q = jnp.maximum(x, s)
    x4 = x.reshape(_H // 4, 4, _W)
    s4 = s.reshape(_H // 4, 4, _W)
    q4 = q.reshape(_H // 4, 4, _W)
    u0 = jnp.maximum(jnp.maximum(q4[:, 0], s4[:, 1]), s4[:, 2])
    u1 = jnp.maximum(jnp.maximum(x4[:, 1], x4[:, 2]), q4[:, 3])
    u = jnp.stack([u0, u1], axis=1).reshape(_H // 2, _W)
    o_ref[0] = jax.lax.dot(u, sel_ref[...],
                           preferred_element_type=jnp.float32)


def _tc_pool(x):
    sel = np.zeros((_W, _W // 2), np.float32)
    sel[np.arange(0, _W, 2), np.arange(_W // 2)] = 1.0
    return pl.pallas_call(
        _tc_body,
        grid=(_P - _PS,),
        in_specs=[
            pl.BlockSpec((1, _H, _W), lambda p: (p + _PS, 0, 0)),
            pl.BlockSpec((_W, _W // 2), lambda p: (0, 0)),
        ],
        out_specs=pl.BlockSpec((1, _H // 2, _W // 2), lambda p: (p, 0, 0)),
        out_shape=jax.ShapeDtypeStruct((_P - _PS, _H // 2, _W // 2),
                                       jnp.float32),
    )(x, jnp.asarray(sel))


def kernel(input, level):
    del level
    x3 = input.reshape(_P, _H, _W)
    sc_out = _sc_pool(x3.reshape(_P * _H * _W))
    tc_out = _tc_pool(x3)
    out = jnp.concatenate(
        [sc_out.reshape(_PS, _H // 2, _W // 2), tc_out], axis=0)
    return out.reshape(_B, _C, _H // 2, _W // 2)


# speedup vs baseline: 2.3110x; 2.3110x over previous
import jax
import jax.numpy as jnp
from jax import lax
from jax.experimental import pallas as pl
from jax.experimental.pallas import tpu as pltpu
from jax.experimental.pallas import tpu_sc as plsc

_NC = 2
_NS = 16
_NW = _NC * _NS
_L = 16

_B, _C, _H, _W = 8, 96, 512, 128
_P = _B * _C
_PPW = _P // _NW
_HALF = 32768
_OHALF = 8192
_NH = 2 * _PPW
_GROUPS = _HALF // 512


def _half_stencil(in_v, out_v):
    iota = lax.iota(jnp.int32, _L)
    ev = iota * 2

    consts = []
    for jb in range(4):
        ce = ev + (32 * jb)
        co = ce + 1
        consts.append((ce, co, co + 128, co + 256,
                       ce + 128, ce + 256, ce + 384, co + 384))

    @plsc.parallel_loop(0, _GROUPS, unroll=2)
    def row_body(i):
        grp = in_v.at[pl.ds(i * 512, 512)]
        for jb in range(4):
            ce, co, co1, co2, ce1, ce2, ce3, co3 = consts[jb]
            a = plsc.load_gather(grp, [ce])
            b = plsc.load_gather(grp, [co])
            c = plsc.load_gather(grp, [co1])
            d = plsc.load_gather(grp, [co2])
            out_v[pl.ds(i * 128 + 16 * jb, _L)] = jnp.maximum(
                jnp.maximum(a, b), jnp.maximum(c, d))
            e = plsc.load_gather(grp, [ce1])
            f = plsc.load_gather(grp, [ce2])
            g = plsc.load_gather(grp, [ce3])
            h = plsc.load_gather(grp, [co3])
            out_v[pl.ds(i * 128 + 64 + 16 * jb, _L)] = jnp.maximum(
                jnp.maximum(e, f), jnp.maximum(g, h))


def _sc_body(in_hbm, out_hbm, in_bufs, out_bufs, in_sems, out_sems):
    wid = lax.axis_index("s") * _NC + lax.axis_index("c")
    ibase = wid * (_NH * _HALF)
    obase = wid * (_NH * _OHALF)

    def in_slice(g):
        return in_hbm.at[pl.ds(ibase + g * _HALF, _HALF)]

    def out_slice(g):
        return out_hbm.at[pl.ds(obase + g * _OHALF, _OHALF)]

    for b in range(2):
        pltpu.async_copy(in_slice(b), in_bufs[b], in_sems[b])

    def pair_body(g2, carry):
        for b in range(2):
            g = g2 * 2 + b
            in_v, out_v = in_bufs[b], out_bufs[b]
            pltpu.make_async_copy(in_slice(g), in_v, in_sems[b]).wait()

            @pl.when(g2 > 0)
            def _wait_out():
                pltpu.make_async_copy(out_v, out_slice(g - 2),
                                      out_sems[b]).wait()

            _half_stencil(in_v, out_v)
            pltpu.async_copy(out_v, out_slice(g), out_sems[b])

            @pl.when(g2 < _NH // 2 - 1)
            def _prefetch():
                pltpu.async_copy(in_slice(g + 2), in_v, in_sems[b])
        return carry

    lax.fori_loop(0, _NH // 2, pair_body, 0)

    for b in range(2):
        pltpu.make_async_copy(out_bufs[b], out_slice(_NH - 2 + b),
                              out_sems[b]).wait()


_sc_pool = pl.kernel(
    _sc_body,
    out_type=jax.ShapeDtypeStruct((_P * _H * _W // 4,), jnp.float32),
    mesh=plsc.VectorSubcoreMesh(core_axis_name="c", subcore_axis_name="s"),
    scratch_types=[
        [pltpu.VMEM((_HALF,), jnp.float32) for _ in range(2)],
        [pltpu.VMEM((_OHALF,), jnp.float32) for _ in range(2)],
        [pltpu.SemaphoreType.DMA for _ in range(2)],
        [pltpu.SemaphoreType.DMA for _ in range(2)],
    ],
    compiler_params=pltpu.CompilerParams(needs_layout_passes=False),
)


def kernel(input, level):
    del level
    x = input.reshape(_P * _H * _W)
    out = _sc_pool(x)
    return out.reshape(_B, _C, _H // 2, _W // 2)


# speedup vs baseline: 2.3405x; 1.0128x over previous
import jax
import jax.numpy as jnp
from jax import lax
from jax.experimental import pallas as pl
from jax.experimental.pallas import tpu as pltpu
from jax.experimental.pallas import tpu_sc as plsc

_NC = 2
_NS = 16
_NW = _NC * _NS
_L = 16

_B, _C, _H, _W = 8, 96, 512, 128
_P = _B * _C
_PPW = _P // _NW
_HALF = 32768
_OHALF = 8192
_NH = 2 * _PPW
_GROUPS = _HALF // 512


def _half_stencil(in_v, out_v):
    iota = lax.iota(jnp.int32, _L)
    ev = iota * 2

    consts = []
    for jb in range(4):
        ce = ev + (32 * jb)
        co = ce + 1
        consts.append((ce, co, co + 128, co + 256,
                       ce + 128, ce + 256, ce + 384, co + 384))

    @plsc.parallel_loop(0, _GROUPS, unroll=2)
    def row_body(i):
        grp = in_v.at[pl.ds(i * 512, 512)]
        for jb in range(4):
            ce, co, co1, co2, ce1, ce2, ce3, co3 = consts[jb]
            a = plsc.load_gather(grp, [ce])
            b = plsc.load_gather(grp, [co])
            c = plsc.load_gather(grp, [co1])
            d = plsc.load_gather(grp, [co2])
            out_v[pl.ds(i * 128 + 16 * jb, _L)] = jnp.maximum(
                jnp.maximum(a, b), jnp.maximum(c, d))
            e = plsc.load_gather(grp, [ce1])
            f = plsc.load_gather(grp, [ce2])
            g = plsc.load_gather(grp, [ce3])
            h = plsc.load_gather(grp, [co3])
            out_v[pl.ds(i * 128 + 64 + 16 * jb, _L)] = jnp.maximum(
                jnp.maximum(e, f), jnp.maximum(g, h))


def _sc_body(in_hbm, out_hbm, in_bufs, out_bufs, in_sems, out_sems):
    wid = lax.axis_index("s") * _NC + lax.axis_index("c")
    ibase = wid * (_NH * _HALF)
    pbase = wid * _PPW

    def in_slice(k, b):
        return in_hbm.at[pl.ds(ibase + (2 * k + b) * _HALF, _HALF)]

    def out_slice(k, b):
        return out_hbm.at[pbase + k, pl.ds(b * _OHALF, _OHALF)]

    for b in range(2):
        pltpu.async_copy(in_slice(0, b), in_bufs[b], in_sems[b])

    def pair_body(k, carry):
        for b in range(2):
            in_v, out_v = in_bufs[b], out_bufs[b]
            pltpu.make_async_copy(in_slice(k, b), in_v, in_sems[b]).wait()

            @pl.when(k > 0)
            def _wait_out():
                pltpu.make_async_copy(out_v, out_slice(k - 1, b),
                                      out_sems[b]).wait()

            _half_stencil(in_v, out_v)
            pltpu.async_copy(out_v, out_slice(k, b), out_sems[b])

            @pl.when(k < _PPW - 1)
            def _prefetch():
                pltpu.async_copy(in_slice(k + 1, b), in_v, in_sems[b])
        return carry

    lax.fori_loop(0, _PPW, pair_body, 0)

    for b in range(2):
        pltpu.make_async_copy(out_bufs[b], out_slice(_PPW - 1, b),
                              out_sems[b]).wait()


_sc_pool = pl.kernel(
    _sc_body,
    out_type=jax.ShapeDtypeStruct((_P, _H * _W // 4), jnp.float32),
    mesh=plsc.VectorSubcoreMesh(core_axis_name="c", subcore_axis_name="s"),
    scratch_types=[
        [pltpu.VMEM((_HALF,), jnp.float32) for _ in range(2)],
        [pltpu.VMEM((_OHALF,), jnp.float32) for _ in range(2)],
        [pltpu.SemaphoreType.DMA for _ in range(2)],
        [pltpu.SemaphoreType.DMA for _ in range(2)],
    ],
    compiler_params=pltpu.CompilerParams(needs_layout_passes=False),
)


def kernel(input, level):
    del level
    x = input.reshape(_P * _H * _W)
    out = _sc_pool(x)
    return out.reshape(_B, _C, _H // 2, _W // 2)


# speedup vs baseline: 3.1500x; 1.3458x over previous
import jax
import jax.numpy as jnp
from jax import lax
from jax.experimental import pallas as pl
from jax.experimental.pallas import tpu as pltpu
from jax.experimental.pallas import tpu_sc as plsc

_NC = 2
_NS = 16
_NW = _NC * _NS
_L = 16

_B, _C, _H, _W = 8, 96, 512, 128
_P = _B * _C
_PPW = _P // _NW
_HALF = 32768
_OHALF = 8192
_NH = 2 * _PPW
_GROUPS = _HALF // 512


def _half_stencil(in_v, out_v):
    iota = lax.iota(jnp.int32, _L)
    ev = iota * 2

    consts = []
    for jb in range(4):
        ce = ev + (32 * jb)
        co = ce + 1
        consts.append((ce, co, co + 128, co + 256,
                       ce + 128, ce + 256, ce + 384, co + 384))

    @plsc.parallel_loop(0, _GROUPS, unroll=2)
    def row_body(i):
        grp = in_v.at[pl.ds(i * 512, 512)]
        for jb in range(4):
            ce, co, co1, co2, ce1, ce2, ce3, co3 = consts[jb]
            a = plsc.load_gather(grp, [ce])
            b = plsc.load_gather(grp, [co])
            c = plsc.load_gather(grp, [co1])
            d = plsc.load_gather(grp, [co2])
            out_v[2 * i, pl.ds(16 * jb, _L)] = jnp.maximum(
                jnp.maximum(a, b), jnp.maximum(c, d))
            e = plsc.load_gather(grp, [ce1])
            f = plsc.load_gather(grp, [ce2])
            g = plsc.load_gather(grp, [ce3])
            h = plsc.load_gather(grp, [co3])
            out_v[2 * i + 1, pl.ds(16 * jb, _L)] = jnp.maximum(
                jnp.maximum(e, f), jnp.maximum(g, h))


def _sc_body(in_hbm, out_hbm, in_bufs, out_bufs, in_sems, out_sems):
    wid = lax.axis_index("s") * _NC + lax.axis_index("c")
    ibase = wid * (_NH * _HALF)
    pbase = wid * _PPW

    def in_slice(k, b):
        return in_hbm.at[pl.ds(ibase + (2 * k + b) * _HALF, _HALF)]

    def out_slice(k, b):
        return out_hbm.at[pbase + k, pl.ds(b * 128, 128), :]

    for b in range(2):
        pltpu.async_copy(in_slice(0, b), in_bufs[b], in_sems[b])

    def pair_body(k, carry):
        for b in range(2):
            in_v, out_v = in_bufs[b], out_bufs[b]
            pltpu.make_async_copy(in_slice(k, b), in_v, in_sems[b]).wait()

            @pl.when(k > 0)
            def _wait_out():
                pltpu.make_async_copy(out_v, out_slice(k - 1, b),
                                      out_sems[b]).wait()

            _half_stencil(in_v, out_v)
            pltpu.async_copy(out_v, out_slice(k, b), out_sems[b])

            @pl.when(k < _PPW - 1)
            def _prefetch():
                pltpu.async_copy(in_slice(k + 1, b), in_v, in_sems[b])
        return carry

    lax.fori_loop(0, _PPW, pair_body, 0)

    for b in range(2):
        pltpu.make_async_copy(out_bufs[b], out_slice(_PPW - 1, b),
                              out_sems[b]).wait()


_sc_pool = pl.kernel(
    _sc_body,
    out_type=jax.ShapeDtypeStruct((_P, _H // 2, _W // 2), jnp.float32),
    mesh=plsc.VectorSubcoreMesh(core_axis_name="c", subcore_axis_name="s"),
    scratch_types=[
        [pltpu.VMEM((_HALF,), jnp.float32) for _ in range(2)],
        [pltpu.VMEM((128, _W // 2), jnp.float32) for _ in range(2)],
        [pltpu.SemaphoreType.DMA for _ in range(2)],
        [pltpu.SemaphoreType.DMA for _ in range(2)],
    ],
    compiler_params=pltpu.CompilerParams(needs_layout_passes=False),
)


def kernel(input, level):
    del level
    x = input.reshape(_P * _H * _W)
    out = _sc_pool(x)
    return out.reshape(_B, _C, _H // 2, _W // 2)
